# Initial kernel scaffold; baseline (speedup 1.0000x reference)
import jax
import jax.numpy as jnp
from jax.experimental import pallas as pl


def kernel(feat, edge_index_r0, edge_index_r1, W0, b0, W1, b1, R1_W, R1_b, R2_W, R2_b):
    raise NotImplementedError("write your pallas kernel here")


# speedup vs baseline: 8.2126x; 8.2126x over previous
import functools

import jax
import jax.numpy as jnp
from jax import lax
from jax.experimental import pallas as pl
from jax.experimental.pallas import tpu as pltpu
from jax.experimental.pallas import tpu_sc as plsc

N = 10000
E = 320000
D = 128
NC = 2
NS = 16
EPT = E // NS
B = 80
NB = EPT // B
BG = 40
NBG = EPT // BG
RING = 5
NP = 10240
RPT = NP // NS
CH = 128

_MESH = plsc.VectorSubcoreMesh(core_axis_name="c", subcore_axis_name="s",
                               num_cores=NC, num_subcores=NS)


def _deg_body(src0, dst0, src1, dst1, out_s0, out_d0, out_s1, out_d1,
              ones_v, i0, i1, i2, i3, i4, buf_v,
              smi0, smi1, smi2, smi3, smi4,
              sms0, sms1, sms2, sms3, sms4, hist_s_sh, hist_d_sh):
  c = lax.axis_index("c")
  s = lax.axis_index("s")
  ib = [i0, i1, i2, i3, i4]
  smi = [smi0, smi1, smi2, smi3, smi4]
  sms = [sms0, sms1, sms2, sms3, sms4]

  def zrow(i, carry):
    buf_v[pl.ds(i * 16, 16)] = jnp.zeros((16,), jnp.float32)
    return carry
  lax.fori_loop(0, RPT // 16, zrow, 0)
  pltpu.sync_copy(buf_v, hist_s_sh.at[pl.ds(s * RPT, RPT)])
  pltpu.sync_copy(buf_v, hist_d_sh.at[pl.ds(s * RPT, RPT)])

  def onerow(i, carry):
    ones_v[pl.ds(i * 16, 16)] = jnp.ones((16,), jnp.float32)
    return carry
  lax.fori_loop(0, B // 16, onerow, 0)
  plsc.subcore_barrier()

  def scatter_edges(e_hbm, hist_sh):
    base = s * EPT

    def idx_start(j, b):
      pltpu.make_async_copy(e_hbm.at[pl.ds(base + j * B, B)],
                            ib[b], smi[b]).start()

    def idx_wait(b):
      pltpu.make_async_copy(e_hbm.at[pl.ds(base, B)], ib[b], smi[b]).wait()

    def sc_start(b):
      pltpu.async_copy(ones_v, hist_sh.at[ib[b]], sms[b], add=True)

    def sc_wait(b):
      pltpu.make_async_copy(ones_v, hist_sh.at[ib[0]], sms[b]).wait()

    for j in range(4):
      idx_start(j, j)

    def outer(t, carry):
      for u in range(5):
        j = 5 * t + u
        b = u
        p = (u - 1) % 5
        idx_wait(b)

        @pl.when(j >= 1)
        def _():
          sc_wait(p)
        sc_start(b)

        @pl.when(j + 4 < NB)
        def _():
          idx_start(j + 4, p)
      return carry
    lax.fori_loop(0, NB // 5, outer, 0)
    sc_wait((NB - 1) % 5)

  @pl.when(c == 0)
  def _():
    scatter_edges(src0, hist_s_sh)
    scatter_edges(dst0, hist_d_sh)

  @pl.when(c == 1)
  def _():
    scatter_edges(src1, hist_s_sh)
    scatter_edges(dst1, hist_d_sh)

  plsc.subcore_barrier()

  def writeout(out_s, out_d):
    sl = pl.ds(s * RPT, RPT)
    pltpu.sync_copy(hist_s_sh.at[sl], buf_v)
    pltpu.sync_copy(buf_v, out_s.at[sl])
    pltpu.sync_copy(hist_d_sh.at[sl], buf_v)
    pltpu.sync_copy(buf_v, out_d.at[sl])

  @pl.when(c == 0)
  def _():
    writeout(out_s0, out_d0)

  @pl.when(c == 1)
  def _():
    writeout(out_s1, out_d1)


_deg_call = pl.kernel(
    _deg_body,
    out_type=[jax.ShapeDtypeStruct((NP,), jnp.float32)] * 4,
    mesh=_MESH,
    scratch_types=(
        [pltpu.VMEM((B,), jnp.float32)]
        + [pltpu.VMEM((B,), jnp.int32)] * 5
        + [pltpu.VMEM((RPT,), jnp.float32)]
        + [pltpu.SemaphoreType.DMA] * 10
        + [pltpu.VMEM_SHARED((NP,), jnp.float32)] * 2
    ),
)


def _gs_body(x0, x1, src0, dst0, src1, dst1, agg0, agg1,
             si0, si1, si2, si3, si4, di0, di1, di2, di3, di4,
             ro0, ro1, ro2, ro3, ro4, buf_v,
             ss0, ss1, ss2, ss3, ss4, sd0, sd1, sd2, sd3, sd4,
             sg0, sg1, sg2, sg3, sg4, sc0, sc1, sc2, sc3, sc4, agg_sh):
  c = lax.axis_index("c")
  s = lax.axis_index("s")
  sidx = [si0, si1, si2, si3, si4]
  didx = [di0, di1, di2, di3, di4]
  rows = [ro0, ro1, ro2, ro3, ro4]
  sems = [ss0, ss1, ss2, ss3, ss4]
  semd = [sd0, sd1, sd2, sd3, sd4]
  semg = [sg0, sg1, sg2, sg3, sg4]
  semc = [sc0, sc1, sc2, sc3, sc4]

  def zrow(i, carry):
    for j in range(D // 16):
      buf_v[i, pl.ds(j * 16, 16)] = jnp.zeros((16,), jnp.float32)
    return carry
  lax.fori_loop(0, CH, zrow, 0)
  for k in range(RPT // CH):
    pltpu.sync_copy(buf_v, agg_sh.at[pl.ds(s * RPT + k * CH, CH), :])
  plsc.subcore_barrier()

  def run(x_hbm, src_hbm, dst_hbm):
    base = s * EPT

    def sidx_start(j, b):
      pltpu.make_async_copy(src_hbm.at[pl.ds(base + j * BG, BG)],
                            sidx[b], sems[b]).start()

    def sidx_wait(b):
      pltpu.make_async_copy(src_hbm.at[pl.ds(base, BG)], sidx[b],
                            sems[b]).wait()

    def didx_start(j, b):
      pltpu.make_async_copy(dst_hbm.at[pl.ds(base + j * BG, BG)],
                            didx[b], semd[b]).start()

    def didx_wait(b):
      pltpu.make_async_copy(dst_hbm.at[pl.ds(base, BG)], didx[b],
                            semd[b]).wait()

    def gather_start(b):
      pltpu.async_copy(x_hbm.at[sidx[b]], rows[b], semg[b])

    def gather_wait(b):
      pltpu.make_async_copy(x_hbm.at[sidx[0]], rows[b], semg[b]).wait()

    def sc_start(b):
      pltpu.async_copy(rows[b], agg_sh.at[didx[b]], semc[b], add=True)

    def sc_wait(b):
      pltpu.make_async_copy(rows[b], agg_sh.at[didx[0]], semc[b]).wait()

    for j in range(4):
      sidx_start(j, j)
      didx_start(j, j)
    for j in range(2):
      sidx_wait(j)
      gather_start(j)

    def outer(t, carry):
      for u in range(5):
        j = 5 * t + u
        b = u
        p = (u - 1) % 5
        g = (u + 2) % 5
        gather_wait(b)

        @pl.when(j >= 1)
        def _():
          sc_wait(p)
        didx_wait(b)
        sc_start(b)

        @pl.when(j + 4 < NBG)
        def _():
          sidx_start(j + 4, p)
          didx_start(j + 4, p)

        @pl.when(j + 2 < NBG)
        def _():
          sidx_wait(g)
          gather_start(g)
      return carry
    lax.fori_loop(0, NBG // 5, outer, 0)
    sc_wait((NBG - 1) % 5)

  @pl.when(c == 0)
  def _():
    run(x0, src0, dst0)

  @pl.when(c == 1)
  def _():
    run(x1, src1, dst1)

  plsc.subcore_barrier()

  def wout(out):
    for k in range(RPT // CH):
      sl = pl.ds(s * RPT + k * CH, CH)
      pltpu.sync_copy(agg_sh.at[sl, :], buf_v)
      pltpu.sync_copy(buf_v, out.at[sl, :])

  @pl.when(c == 0)
  def _():
    wout(agg0)

  @pl.when(c == 1)
  def _():
    wout(agg1)


_gs_call = pl.kernel(
    _gs_body,
    out_type=[jax.ShapeDtypeStruct((NP, D), jnp.float32)] * 2,
    mesh=_MESH,
    scratch_types=(
        [pltpu.VMEM((BG,), jnp.int32)] * 10
        + [pltpu.VMEM((BG, D), jnp.float32)] * 5
        + [pltpu.VMEM((CH, D), jnp.float32)]
        + [pltpu.SemaphoreType.DMA] * 20
        + [pltpu.VMEM_SHARED((NP, D), jnp.float32)]
    ),
)

BR = 1000


def _pre_body(feat_ref, w0_ref, w1_ref, hs0_ref, hs1_ref, x0_ref, x1_ref):
  f = feat_ref[...]
  s0 = lax.rsqrt(jnp.maximum(hs0_ref[...], 1.0))
  s1 = lax.rsqrt(jnp.maximum(hs1_ref[...], 1.0))
  x0_ref[...] = jnp.dot(f * s0, w0_ref[...],
                        preferred_element_type=jnp.float32)
  x1_ref[...] = jnp.dot(f * s1, w1_ref[...],
                        preferred_element_type=jnp.float32)


def _post_body(a0_ref, a1_ref, hd0_ref, hd1_ref, b0_ref, b1_ref,
               r1w_ref, r1b_ref, r2w_ref, r2b_ref,
               rec_ref, h0_ref, h1_ref):
  s0 = lax.rsqrt(jnp.maximum(hd0_ref[...], 1.0))
  s1 = lax.rsqrt(jnp.maximum(hd1_ref[...], 1.0))
  h0 = jnp.maximum(a0_ref[...] * s0 + b0_ref[...], 0.0)
  h1 = jnp.maximum(a1_ref[...] * s1 + b1_ref[...], 0.0)
  h0_ref[...] = h0
  h1_ref[...] = h1
  hid = jnp.maximum(
      jnp.dot(h0 + h1, r1w_ref[...], preferred_element_type=jnp.float32)
      + r1b_ref[...], 0.0)
  rec_ref[...] = (jnp.dot(hid, r2w_ref[...], preferred_element_type=jnp.float32)
                  + r2b_ref[...])


def _row_spec(w):
  return pl.BlockSpec((BR, w), lambda i: (i, 0))


def _full_spec(h, w):
  return pl.BlockSpec((h, w), lambda i: (0, 0))


_pre_call = pl.pallas_call(
    _pre_body,
    grid=(N // BR,),
    in_specs=[_row_spec(D), _full_spec(D, D), _full_spec(D, D),
              _row_spec(1), _row_spec(1)],
    out_specs=[_row_spec(D), _row_spec(D)],
    out_shape=[jax.ShapeDtypeStruct((N, D), jnp.float32)] * 2,
)

_post_call = pl.pallas_call(
    _post_body,
    grid=(N // BR,),
    in_specs=[_row_spec(D), _row_spec(D), _row_spec(1), _row_spec(1),
              _full_spec(1, D), _full_spec(1, D),
              _full_spec(D, D), _full_spec(1, D),
              _full_spec(D, D), _full_spec(1, D)],
    out_specs=[_row_spec(D), _row_spec(D), _row_spec(D)],
    out_shape=[jax.ShapeDtypeStruct((N, D), jnp.float32)] * 3,
)


def kernel(feat, edge_index_r0, edge_index_r1, W0, b0, W1, b1,
           R1_W, R1_b, R2_W, R2_b):
  src0, dst0 = edge_index_r0[0], edge_index_r0[1]
  src1, dst1 = edge_index_r1[0], edge_index_r1[1]
  deg_s0, deg_d0, deg_s1, deg_d1 = _deg_call(src0, dst0, src1, dst1)
  deg_s0 = deg_s0[:N].reshape(N, 1)
  deg_d0 = deg_d0[:N].reshape(N, 1)
  deg_s1 = deg_s1[:N].reshape(N, 1)
  deg_d1 = deg_d1[:N].reshape(N, 1)
  x0, x1 = _pre_call(feat, W0, W1, deg_s0, deg_s1)
  agg0, agg1 = _gs_call(x0, x1, src0, dst0, src1, dst1)
  rec, h0, h1 = _post_call(
      agg0, agg1, deg_d0, deg_d1,
      b0.reshape(1, D), b1.reshape(1, D),
      R1_W, R1_b.reshape(1, D), R2_W, R2_b.reshape(1, D))
  return (rec, h0, h1)
